# Initial kernel scaffold; baseline (speedup 1.0000x reference)
import jax
import jax.numpy as jnp
from jax.experimental import pallas as pl


def kernel(x, w1, b1, w2, b2):
    raise NotImplementedError("write your pallas kernel here")


# speedup vs baseline: 1.2036x; 1.2036x over previous
import jax
import jax.numpy as jnp
from jax.experimental import pallas as pl
from jax.experimental.pallas import tpu as pltpu

_PACK = 8
_BLOCK_ROWS = 2048


def _round_up(n, m):
    return ((n + m - 1) // m) * m


def _mlp_body(x_ref, w1_ref, b1_ref, w2_ref, b2_ref, o_ref):
    xb = x_ref[...].astype(jnp.bfloat16)
    h = jnp.dot(xb, w1_ref[...], preferred_element_type=jnp.float32)
    h = jnp.maximum(h + b1_ref[...], 0.0).astype(jnp.bfloat16)
    y = jnp.dot(h, w2_ref[...], preferred_element_type=jnp.float32)
    o_ref[...] = y + b2_ref[...]


def kernel(x, w1, b1, w2, b2):
    B, K = x.shape
    H = w1.shape[1]
    N = w2.shape[1]

    x = x.astype(jnp.float32)
    w1 = w1.astype(jnp.float32)
    w2 = w2.astype(jnp.float32)
    b1 = b1.reshape(1, H).astype(jnp.float32)
    b2 = b2.reshape(1, N).astype(jnp.float32)

    P = _PACK
    while P > 1 and B % P:
        P //= 2
    Bp = B // P

    eye = jnp.eye(P, dtype=jnp.float32)
    w1bd = jnp.kron(eye, w1).astype(jnp.bfloat16)
    w2bd = jnp.kron(eye, w2).astype(jnp.bfloat16)
    b1t = jnp.tile(b1, (1, P))
    b2t = jnp.tile(b2, (1, P))
    xp = x.reshape(Bp, P * K)

    block_r = min(_BLOCK_ROWS, _round_up(Bp, 8))
    grid_r = pl.cdiv(Bp, block_r)

    out = pl.pallas_call(
        _mlp_body,
        out_shape=jax.ShapeDtypeStruct((Bp, P * N), jnp.float32),
        grid_spec=pltpu.PrefetchScalarGridSpec(
            num_scalar_prefetch=0,
            grid=(grid_r,),
            in_specs=[
                pl.BlockSpec((block_r, P * K), lambda i: (i, 0)),
                pl.BlockSpec((P * K, P * H), lambda i: (0, 0)),
                pl.BlockSpec((1, P * H), lambda i: (0, 0)),
                pl.BlockSpec((P * H, P * N), lambda i: (0, 0)),
                pl.BlockSpec((1, P * N), lambda i: (0, 0)),
            ],
            out_specs=pl.BlockSpec((block_r, P * N), lambda i: (i, 0)),
        ),
        compiler_params=pltpu.CompilerParams(
            dimension_semantics=("parallel",)),
    )(xp, w1bd, b1t, w2bd, b2t)

    return out.reshape(B, N)


# speedup vs baseline: 1.2430x; 1.0327x over previous
import jax
import jax.numpy as jnp
from jax.experimental import pallas as pl
from jax.experimental.pallas import tpu as pltpu

_PACK = 8
_BLOCK_ROWS = 8192


def _round_up(n, m):
    return ((n + m - 1) // m) * m


def _mlp_body(x_ref, w1_ref, b1_ref, w2_ref, b2_ref, o_ref):
    xb = x_ref[...].astype(jnp.bfloat16)
    h = jnp.dot(xb, w1_ref[...], preferred_element_type=jnp.float32)
    h = jnp.maximum(h + b1_ref[...], 0.0).astype(jnp.bfloat16)
    y = jnp.dot(h, w2_ref[...], preferred_element_type=jnp.float32)
    o_ref[...] = y + b2_ref[...]


def kernel(x, w1, b1, w2, b2):
    B, K = x.shape
    H = w1.shape[1]
    N = w2.shape[1]

    x = x.astype(jnp.float32)
    w1 = w1.astype(jnp.float32)
    w2 = w2.astype(jnp.float32)
    b1 = b1.reshape(1, H).astype(jnp.float32)
    b2 = b2.reshape(1, N).astype(jnp.float32)

    P = _PACK
    while P > 1 and B % P:
        P //= 2
    Bp = B // P

    eye = jnp.eye(P, dtype=jnp.float32)
    w1bd = jnp.kron(eye, w1).astype(jnp.bfloat16)
    w2bd = jnp.kron(eye, w2).astype(jnp.bfloat16)
    b1t = jnp.tile(b1, (1, P))
    b2t = jnp.tile(b2, (1, P))
    xp = x.reshape(Bp, P * K)

    block_r = min(_BLOCK_ROWS, _round_up(Bp, 8))
    grid_r = pl.cdiv(Bp, block_r)

    out = pl.pallas_call(
        _mlp_body,
        out_shape=jax.ShapeDtypeStruct((Bp, P * N), jnp.float32),
        grid_spec=pltpu.PrefetchScalarGridSpec(
            num_scalar_prefetch=0,
            grid=(grid_r,),
            in_specs=[
                pl.BlockSpec((block_r, P * K), lambda i: (i, 0)),
                pl.BlockSpec((P * K, P * H), lambda i: (0, 0)),
                pl.BlockSpec((1, P * H), lambda i: (0, 0)),
                pl.BlockSpec((P * H, P * N), lambda i: (0, 0)),
                pl.BlockSpec((1, P * N), lambda i: (0, 0)),
            ],
            out_specs=pl.BlockSpec((block_r, P * N), lambda i: (i, 0)),
        ),
        compiler_params=pltpu.CompilerParams(
            dimension_semantics=("parallel",)),
    )(xp, w1bd, b1t, w2bd, b2t)

    return out.reshape(B, N)


# speedup vs baseline: 1.3051x; 1.0499x over previous
import jax
import jax.numpy as jnp
from jax.experimental import pallas as pl
from jax.experimental.pallas import tpu as pltpu

_BLOCK_ROWS = 16384


def _round_up(n, m):
    return ((n + m - 1) // m) * m


def _mlp_body(x_ref, w1_ref, b1_ref, w2_ref, b2_ref, o_ref):
    h = jnp.dot(x_ref[...], w1_ref[...], preferred_element_type=jnp.float32)
    h = jnp.maximum(h + b1_ref[...], 0.0)
    y = jnp.dot(h, w2_ref[...], preferred_element_type=jnp.float32)
    o_ref[...] = y + b2_ref[...]


def kernel(x, w1, b1, w2, b2):
    B, K = x.shape
    H = w1.shape[1]
    N = w2.shape[1]

    x = x.astype(jnp.float32)
    w1 = w1.astype(jnp.float32)
    w2 = w2.astype(jnp.float32)
    b1 = b1.reshape(1, H).astype(jnp.float32)
    b2 = b2.reshape(1, N).astype(jnp.float32)

    block_b = min(_BLOCK_ROWS, _round_up(B, 8))
    grid_b = pl.cdiv(B, block_b)

    return pl.pallas_call(
        _mlp_body,
        out_shape=jax.ShapeDtypeStruct((B, N), jnp.float32),
        grid_spec=pltpu.PrefetchScalarGridSpec(
            num_scalar_prefetch=0,
            grid=(grid_b,),
            in_specs=[
                pl.BlockSpec((block_b, K), lambda i: (i, 0)),
                pl.BlockSpec((K, H), lambda i: (0, 0)),
                pl.BlockSpec((1, H), lambda i: (0, 0)),
                pl.BlockSpec((H, N), lambda i: (0, 0)),
                pl.BlockSpec((1, N), lambda i: (0, 0)),
            ],
            out_specs=pl.BlockSpec((block_b, N), lambda i: (i, 0)),
        ),
        compiler_params=pltpu.CompilerParams(
            dimension_semantics=("parallel",)),
    )(x, w1, b1, w2, b2)


# speedup vs baseline: 15.1429x; 11.6031x over previous
import jax
import jax.numpy as jnp
from jax.experimental import pallas as pl
from jax.experimental.pallas import tpu as pltpu

_BLOCK_COLS = 16384


def _round_up(n, m):
    return ((n + m - 1) // m) * m


def _mlp_body(xt_ref, w1_ref, b1_ref, w2_ref, b2_ref, o_ref):
    xb = xt_ref[...].astype(jnp.bfloat16)
    w1 = w1_ref[...].astype(jnp.bfloat16)
    w2 = w2_ref[...].astype(jnp.bfloat16)
    ht = jax.lax.dot_general(w1, xb, (((0,), (0,)), ((), ())),
                             preferred_element_type=jnp.float32)
    ht = jnp.maximum(ht + b1_ref[...], 0.0).astype(jnp.bfloat16)
    yt = jax.lax.dot_general(w2_ref[...].astype(jnp.bfloat16), ht,
                             (((0,), (0,)), ((), ())),
                             preferred_element_type=jnp.float32)
    o_ref[...] = yt + b2_ref[...]


def kernel(x, w1, b1, w2, b2):
    B, K = x.shape
    H = w1.shape[1]
    N = w2.shape[1]

    x = x.astype(jnp.float32)
    w1 = w1.astype(jnp.float32)
    w2 = w2.astype(jnp.float32)
    b1c = b1.reshape(H, 1).astype(jnp.float32)
    b2c = b2.reshape(N, 1).astype(jnp.float32)

    xt = x.T

    block_c = min(_BLOCK_COLS, _round_up(B, 128))
    grid_c = pl.cdiv(B, block_c)

    yt = pl.pallas_call(
        _mlp_body,
        out_shape=jax.ShapeDtypeStruct((N, B), jnp.float32),
        grid_spec=pltpu.PrefetchScalarGridSpec(
            num_scalar_prefetch=0,
            grid=(grid_c,),
            in_specs=[
                pl.BlockSpec((K, block_c), lambda i: (0, i)),
                pl.BlockSpec((K, H), lambda i: (0, 0)),
                pl.BlockSpec((H, 1), lambda i: (0, 0)),
                pl.BlockSpec((H, N), lambda i: (0, 0)),
                pl.BlockSpec((N, 1), lambda i: (0, 0)),
            ],
            out_specs=pl.BlockSpec((N, block_c), lambda i: (0, i)),
        ),
        compiler_params=pltpu.CompilerParams(
            dimension_semantics=("parallel",)),
    )(xt, w1, b1c, w2, b2c)

    return yt.T


# speedup vs baseline: 24.2330x; 1.6003x over previous
import jax
import jax.numpy as jnp
from jax.experimental import pallas as pl
from jax.experimental.pallas import tpu as pltpu

_BLOCK_COLS = 65536


def _round_up(n, m):
    return ((n + m - 1) // m) * m


def _mlp_body(xt_ref, w1_ref, b1_ref, w2_ref, b2_ref, o_ref):
    xb = xt_ref[...].astype(jnp.bfloat16)
    w1 = w1_ref[...].astype(jnp.bfloat16)
    w2 = w2_ref[...].astype(jnp.bfloat16)
    ht = jax.lax.dot_general(w1, xb, (((0,), (0,)), ((), ())),
                             preferred_element_type=jnp.float32)
    ht = jnp.maximum(ht + b1_ref[...], 0.0).astype(jnp.bfloat16)
    yt = jax.lax.dot_general(w2_ref[...].astype(jnp.bfloat16), ht,
                             (((0,), (0,)), ((), ())),
                             preferred_element_type=jnp.float32)
    o_ref[...] = yt + b2_ref[...]


def kernel(x, w1, b1, w2, b2):
    B, K = x.shape
    H = w1.shape[1]
    N = w2.shape[1]

    x = x.astype(jnp.float32)
    w1 = w1.astype(jnp.float32)
    w2 = w2.astype(jnp.float32)
    b1c = b1.reshape(H, 1).astype(jnp.float32)
    b2c = b2.reshape(N, 1).astype(jnp.float32)

    xt = x.T

    block_c = min(_BLOCK_COLS, _round_up(B, 128))
    grid_c = pl.cdiv(B, block_c)

    yt = pl.pallas_call(
        _mlp_body,
        out_shape=jax.ShapeDtypeStruct((N, B), jnp.float32),
        grid_spec=pltpu.PrefetchScalarGridSpec(
            num_scalar_prefetch=0,
            grid=(grid_c,),
            in_specs=[
                pl.BlockSpec((K, block_c), lambda i: (0, i)),
                pl.BlockSpec((K, H), lambda i: (0, 0)),
                pl.BlockSpec((H, 1), lambda i: (0, 0)),
                pl.BlockSpec((H, N), lambda i: (0, 0)),
                pl.BlockSpec((N, 1), lambda i: (0, 0)),
            ],
            out_specs=pl.BlockSpec((N, block_c), lambda i: (0, i)),
        ),
        compiler_params=pltpu.CompilerParams(
            dimension_semantics=("parallel",)),
    )(xt, w1, b1c, w2, b2c)

    return yt.T


# speedup vs baseline: 26.3312x; 1.0866x over previous
import jax
import jax.numpy as jnp
from jax.experimental import pallas as pl
from jax.experimental.pallas import tpu as pltpu

_BLOCK_COLS = 131072


def _round_up(n, m):
    return ((n + m - 1) // m) * m


def _mlp_body(xt_ref, w1_ref, b1_ref, w2_ref, b2_ref, o_ref):
    xb = xt_ref[...].astype(jnp.bfloat16)
    w1 = w1_ref[...].astype(jnp.bfloat16)
    w2 = w2_ref[...].astype(jnp.bfloat16)
    ht = jax.lax.dot_general(w1, xb, (((0,), (0,)), ((), ())),
                             preferred_element_type=jnp.float32)
    ht = jnp.maximum(ht + b1_ref[...], 0.0).astype(jnp.bfloat16)
    yt = jax.lax.dot_general(w2_ref[...].astype(jnp.bfloat16), ht,
                             (((0,), (0,)), ((), ())),
                             preferred_element_type=jnp.float32)
    o_ref[...] = yt + b2_ref[...]


def kernel(x, w1, b1, w2, b2):
    B, K = x.shape
    H = w1.shape[1]
    N = w2.shape[1]

    x = x.astype(jnp.float32)
    w1 = w1.astype(jnp.float32)
    w2 = w2.astype(jnp.float32)
    b1c = b1.reshape(H, 1).astype(jnp.float32)
    b2c = b2.reshape(N, 1).astype(jnp.float32)

    xt = x.T

    block_c = min(_BLOCK_COLS, _round_up(B, 128))
    grid_c = pl.cdiv(B, block_c)

    yt = pl.pallas_call(
        _mlp_body,
        out_shape=jax.ShapeDtypeStruct((N, B), jnp.float32),
        grid_spec=pltpu.PrefetchScalarGridSpec(
            num_scalar_prefetch=0,
            grid=(grid_c,),
            in_specs=[
                pl.BlockSpec((K, block_c), lambda i: (0, i)),
                pl.BlockSpec((K, H), lambda i: (0, 0)),
                pl.BlockSpec((H, 1), lambda i: (0, 0)),
                pl.BlockSpec((H, N), lambda i: (0, 0)),
                pl.BlockSpec((N, 1), lambda i: (0, 0)),
            ],
            out_specs=pl.BlockSpec((N, block_c), lambda i: (0, i)),
        ),
        compiler_params=pltpu.CompilerParams(
            dimension_semantics=("parallel",)),
    )(xt, w1, b1c, w2, b2c)

    return yt.T


# speedup vs baseline: 28.0041x; 1.0635x over previous
import jax
import jax.numpy as jnp
from jax.experimental import pallas as pl
from jax.experimental.pallas import tpu as pltpu

_BLOCK_COLS = 131072


def _round_up(n, m):
    return ((n + m - 1) // m) * m


def _mlp_body(xt_ref, w1_ref, b1_ref, w2_ref, b2_ref, o_ref):
    xb = xt_ref[...].astype(jnp.bfloat16)
    w1 = w1_ref[...].astype(jnp.bfloat16)
    w2 = w2_ref[...].astype(jnp.bfloat16)
    b1c = b1_ref[...].T
    b2c = b2_ref[...].T
    ht = jax.lax.dot_general(w1, xb, (((0,), (0,)), ((), ())),
                             preferred_element_type=jnp.float32)
    ht = jnp.maximum(ht + b1c, 0.0).astype(jnp.bfloat16)
    yt = jax.lax.dot_general(w2, ht, (((0,), (0,)), ((), ())),
                             preferred_element_type=jnp.float32)
    o_ref[...] = yt + b2c


def kernel(x, w1, b1, w2, b2):
    B, K = x.shape
    H = w1.shape[1]
    N = w2.shape[1]

    x = x.astype(jnp.float32)
    w1 = w1.astype(jnp.float32)
    w2 = w2.astype(jnp.float32)
    b1r = b1.reshape(1, H).astype(jnp.float32)
    b2r = b2.reshape(1, N).astype(jnp.float32)

    xt = x.T

    block_c = min(_BLOCK_COLS, _round_up(B, 128))
    grid_c = pl.cdiv(B, block_c)

    yt = pl.pallas_call(
        _mlp_body,
        out_shape=jax.ShapeDtypeStruct((N, B), jnp.float32),
        grid_spec=pltpu.PrefetchScalarGridSpec(
            num_scalar_prefetch=0,
            grid=(grid_c,),
            in_specs=[
                pl.BlockSpec((K, block_c), lambda i: (0, i)),
                pl.BlockSpec((K, H), lambda i: (0, 0)),
                pl.BlockSpec((1, H), lambda i: (0, 0)),
                pl.BlockSpec((H, N), lambda i: (0, 0)),
                pl.BlockSpec((1, N), lambda i: (0, 0)),
            ],
            out_specs=pl.BlockSpec((N, block_c), lambda i: (0, i)),
        ),
        compiler_params=pltpu.CompilerParams(
            dimension_semantics=("parallel",)),
    )(xt, w1, b1r, w2, b2r)

    return yt.T
